# baseline (speedup 1.0000x reference)
import functools

import jax
import jax.numpy as jnp
from jax import lax
from jax.experimental import pallas as pl
from jax.experimental.pallas import tpu as pltpu
from jax.experimental.pallas import tpu_sc as plsc

NC = 2
NS = 16
NW = NC * NS

VOCAB = 1_000_000
D = 64
B_TOTAL = 16384 * 50
B_PER_W = B_TOTAL // NW

SIDX = 32
SN = 10
NG = 5
CHUNK = SN * SIDX
NCHUNK = B_PER_W // CHUNK
NROW = B_PER_W // SIDX

assert B_PER_W % CHUNK == 0
assert NCHUNK % NG == 0 and NCHUNK > 2 * NG


def _sc_gather(table, idx2d):
    mesh = plsc.VectorSubcoreMesh(
        core_axis_name="c", subcore_axis_name="s", num_cores=NC, num_subcores=NS
    )

    @functools.partial(
        pl.kernel,
        out_type=jax.ShapeDtypeStruct((B_TOTAL, D), jnp.float32),
        mesh=mesh,
        scratch_types=[
            pltpu.VMEM((NROW, SIDX), jnp.int32),
        ]
        + [pltpu.VMEM((CHUNK, D), jnp.float32) for _ in range(NG)]
        + [pltpu.SemaphoreType.DMA for _ in range(2 * NG)],
        compiler_params=pltpu.CompilerParams(use_tc_tiling_on_sc=False),
    )
    def k(table_hbm, idx_hbm, out_hbm, idx_v, *bufs_and_sems):
        rows = bufs_and_sems[:NG]
        gsem = bufs_and_sems[NG:2 * NG]
        wsem = bufs_and_sems[2 * NG:]
        wid = lax.axis_index("s") * NC + lax.axis_index("c")
        base = wid * B_PER_W

        pltpu.sync_copy(idx_hbm.at[pl.ds(wid * NROW, NROW)], idx_v)

        def start_gather(c, s):
            for j in range(SN):
                pltpu.async_copy(
                    table_hbm.at[idx_v.at[c * SN + j]],
                    rows[s].at[pl.ds(j * SIDX, SIDX)],
                    gsem[s],
                )

        def wait_gather(s):
            pltpu.make_async_copy(
                table_hbm.at[pl.ds(0, CHUNK)], rows[s], gsem[s]
            ).wait()

        def start_wb(c, s):
            pltpu.async_copy(
                rows[s], out_hbm.at[pl.ds(base + c * CHUNK, CHUNK)], wsem[s]
            )

        def wait_wb(s):
            pltpu.make_async_copy(
                rows[s], out_hbm.at[pl.ds(0, CHUNK)], wsem[s]
            ).wait()

        for c in range(NG - 1):
            start_gather(c, c)
        start_gather(NG - 1, NG - 1)
        wait_gather(0)
        start_wb(0, 0)

        def body(i, carry):
            c0 = 1 + i * NG
            for d in range(NG):
                c = c0 + d
                wait_wb(d)
                start_gather(c + NG - 1, d)
                wait_gather((d + 1) % NG)
                start_wb(c, (d + 1) % NG)
            return carry

        lax.fori_loop(0, (NCHUNK - NG) // NG, body, 0)

        for c in range(NCHUNK - NG + 1, NCHUNK):
            wait_gather(c % NG)
            start_wb(c, c % NG)
        for c in range(NCHUNK - NG, NCHUNK):
            wait_wb(c % NG)

    return k(table, idx2d)


def kernel(indices, table):
    idx2d = indices.astype(jnp.int32).reshape(B_TOTAL // SIDX, SIDX)
    out = _sc_gather(table, idx2d)
    return out.reshape(indices.shape[0], indices.shape[1], D)
